# Initial kernel scaffold; baseline (speedup 1.0000x reference)
import functools
import math

import jax
import jax.numpy as jnp
from jax import lax
from jax.experimental import pallas as pl
from jax.experimental.pallas import tpu as pltpu
from jax.experimental.pallas import tpu_sc as plsc

NW = 32
NSUB = 16
K = 80


def _sc_deg(dst3, n):
  nch = dst3.shape[1]
  zeros = jnp.zeros((n, 16), jnp.float32)
  ones = jnp.ones((K, 16), jnp.float32)
  rows_pt = n // NSUB
  mesh = plsc.VectorSubcoreMesh(core_axis_name="c", subcore_axis_name="s")

  @functools.partial(
      pl.kernel,
      mesh=mesh,
      out_type=jax.ShapeDtypeStruct((2, n, 16), jnp.float32),
      scratch_types=[
          pltpu.VMEM((nch, K), jnp.int32),
          pltpu.VMEM((K, 16), jnp.float32),
          pltpu.VMEM_SHARED((n, 16), jnp.float32),
      ],
  )
  def deg_kernel(dst_hbm, zeros_hbm, ones_hbm, out_hbm, dst_v, ones_v, acc):
    c = lax.axis_index("c")
    s = lax.axis_index("s")
    wid = c * NSUB + s
    pltpu.sync_copy(dst_hbm.at[wid], dst_v)
    pltpu.sync_copy(ones_hbm, ones_v)
    base = s * rows_pt
    pltpu.sync_copy(zeros_hbm.at[pl.ds(base, rows_pt)],
                    acc.at[pl.ds(base, rows_pt)])
    plsc.subcore_barrier()

    @pl.loop(0, nch)
    def _chunk(j):
      pltpu.sync_copy(ones_v, acc.at[dst_v.at[j]], add=True)

    plsc.subcore_barrier()
    pltpu.sync_copy(acc.at[pl.ds(base, rows_pt)],
                    out_hbm.at[c, pl.ds(base, rows_pt)])

  return deg_kernel(dst3, zeros, ones)


def _sc_scatter(g, src3, dst3):
  n, h = g.shape
  nch = src3.shape[1]
  zeros = jnp.zeros((n, h), jnp.float32)
  rows_pt = n // NSUB
  mesh = plsc.VectorSubcoreMesh(core_axis_name="c", subcore_axis_name="s")

  @functools.partial(
      pl.kernel,
      mesh=mesh,
      out_type=jax.ShapeDtypeStruct((2, n, h), jnp.float32),
      scratch_types=[
          pltpu.VMEM((nch, K), jnp.int32),
          pltpu.VMEM((nch, K), jnp.int32),
          pltpu.VMEM((K, h), jnp.float32),
          pltpu.VMEM_SHARED((n, h), jnp.float32),
          pltpu.SemaphoreType.DMA,
      ],
  )
  def scat_kernel(g_hbm, src_hbm, dst_hbm, zeros_hbm, out_hbm,
                  src_v, dst_v, rows_v, acc, sem):
    c = lax.axis_index("c")
    s = lax.axis_index("s")
    wid = c * NSUB + s
    pltpu.sync_copy(src_hbm.at[wid], src_v)
    pltpu.sync_copy(dst_hbm.at[wid], dst_v)
    base = s * rows_pt
    pltpu.sync_copy(zeros_hbm.at[pl.ds(base, rows_pt)],
                    acc.at[pl.ds(base, rows_pt)])
    plsc.subcore_barrier()

    @pl.loop(0, nch)
    def _chunk(j):
      pltpu.async_copy(g_hbm.at[src_v.at[j]], rows_v, sem).wait()
      pltpu.sync_copy(rows_v, acc.at[dst_v.at[j]], add=True)

    plsc.subcore_barrier()
    pltpu.sync_copy(acc.at[pl.ds(base, rows_pt)],
                    out_hbm.at[c, pl.ds(base, rows_pt)])

  return scat_kernel(g, src3, dst3, zeros)


_BLK = 1000


def _embed_body(x_ref, rid_ref, rt_ref, wp_ref, bp_ref, w0_ref, deg_ref,
                g_ref, dis_ref):
  i = pl.program_id(0)
  f = x_ref.shape[1]
  blk = x_ref.shape[0]
  deg = deg_ref[0, :, 0:1] + deg_ref[1, :, 0:1] + 1.0
  dis = lax.rsqrt(deg)
  rtp = jnp.dot(rt_ref[...], wp_ref[f:2 * f, :],
                preferred_element_type=jnp.float32)
  rid = rid_ref[...]
  emb = jnp.zeros((blk, rtp.shape[1]), jnp.float32)
  for r in range(rt_ref.shape[0]):
    emb = emb + jnp.where(rid == r, rtp[r:r + 1, :], 0.0)
  pos = lax.broadcasted_iota(jnp.float32, (blk, f), 0) + (i * blk)
  col = lax.broadcasted_iota(jnp.float32, (blk, f), 1)
  half = jnp.floor(col * 0.5)
  rates = jnp.exp(half * (-2.0 / f * math.log(10000.0)))
  ang = pos * rates
  even = (half * 2.0) == col
  pe = jnp.where(even, jnp.sin(ang), jnp.cos(ang))
  h0 = (jnp.dot(x_ref[...], wp_ref[0:f, :],
                preferred_element_type=jnp.float32)
        + emb
        + jnp.dot(pe, wp_ref[2 * f:3 * f, :],
                  preferred_element_type=jnp.float32)
        + bp_ref[...])
  g_ref[...] = dis * jnp.dot(h0, w0_ref[...],
                             preferred_element_type=jnp.float32)
  dis_ref[...] = dis


def _tc_embed(x, rid2, region_table, Wp, bp2, W0, deg_parts):
  n, f = x.shape
  h = W0.shape[1]
  grid = n // _BLK
  return pl.pallas_call(
      _embed_body,
      grid=(grid,),
      in_specs=[
          pl.BlockSpec((_BLK, f), lambda i: (i, 0)),
          pl.BlockSpec((_BLK, 1), lambda i: (i, 0)),
          pl.BlockSpec(region_table.shape, lambda i: (0, 0)),
          pl.BlockSpec(Wp.shape, lambda i: (0, 0)),
          pl.BlockSpec((1, h), lambda i: (0, 0)),
          pl.BlockSpec((f, h), lambda i: (0, 0)),
          pl.BlockSpec((2, _BLK, 16), lambda i: (0, i, 0)),
      ],
      out_specs=[
          pl.BlockSpec((_BLK, h), lambda i: (i, 0)),
          pl.BlockSpec((_BLK, 1), lambda i: (i, 0)),
      ],
      out_shape=[
          jax.ShapeDtypeStruct((n, h), jnp.float32),
          jax.ShapeDtypeStruct((n, 1), jnp.float32),
      ],
  )(x, rid2, region_table, Wp, bp2, W0, deg_parts)


def _layer_h(s_ref, g_ref, dis_ref, b_ref, gam_ref, bet_ref):
  dis = dis_ref[...]
  a = dis * (s_ref[0] + s_ref[1] + g_ref[...]) + b_ref[...]
  r = jnp.maximum(a, 0.0)
  mu = jnp.mean(r, axis=-1, keepdims=True)
  d = r - mu
  var = jnp.mean(d * d, axis=-1, keepdims=True)
  return d * lax.rsqrt(var + 1e-5) * gam_ref[...] + bet_ref[...]


def _post_body(s_ref, g_ref, dis_ref, b_ref, gam_ref, bet_ref, wn_ref,
               gn_ref):
  hn = _layer_h(s_ref, g_ref, dis_ref, b_ref, gam_ref, bet_ref)
  gn_ref[...] = dis_ref[...] * jnp.dot(hn, wn_ref[...],
                                       preferred_element_type=jnp.float32)


def _tc_post(s_parts, g, dis, b2, gam2, bet2, Wn):
  n, h = g.shape
  grid = n // _BLK
  return pl.pallas_call(
      _post_body,
      grid=(grid,),
      in_specs=[
          pl.BlockSpec((2, _BLK, h), lambda i: (0, i, 0)),
          pl.BlockSpec((_BLK, h), lambda i: (i, 0)),
          pl.BlockSpec((_BLK, 1), lambda i: (i, 0)),
          pl.BlockSpec((1, h), lambda i: (0, 0)),
          pl.BlockSpec((1, h), lambda i: (0, 0)),
          pl.BlockSpec((1, h), lambda i: (0, 0)),
          pl.BlockSpec((h, h), lambda i: (0, 0)),
      ],
      out_specs=pl.BlockSpec((_BLK, h), lambda i: (i, 0)),
      out_shape=jax.ShapeDtypeStruct((n, h), jnp.float32),
  )(s_parts, g, dis, b2, gam2, bet2, Wn)


def _final_body(s_ref, g_ref, dis_ref, b_ref, gam_ref, bet_ref,
                w1_ref, b1_ref, w2_ref, b2_ref, out_ref, acc_ref, *, n):
  i = pl.program_id(0)
  hn = _layer_h(s_ref, g_ref, dis_ref, b_ref, gam_ref, bet_ref)
  part = jnp.sum(hn, axis=0, keepdims=True)

  @pl.when(i == 0)
  def _():
    acc_ref[...] = part

  @pl.when(i > 0)
  def _():
    acc_ref[...] = acc_ref[...] + part

  @pl.when(i == pl.num_programs(0) - 1)
  def _():
    pooled = acc_ref[...] * (1.0 / n)
    z = jnp.dot(pooled, w1_ref[...],
                preferred_element_type=jnp.float32) + b1_ref[...]
    hid = 0.5 * z * (1.0 + lax.erf(z * (1.0 / math.sqrt(2.0))))
    out_ref[...] = jnp.dot(hid, w2_ref[...],
                           preferred_element_type=jnp.float32) + b2_ref[...]


def _tc_final(s_parts, g, dis, b2, gam2, bet2, W1, b12, W2, b22, n):
  h = g.shape[1]
  out_dim = W2.shape[1]
  grid = n // _BLK
  return pl.pallas_call(
      functools.partial(_final_body, n=n),
      grid=(grid,),
      in_specs=[
          pl.BlockSpec((2, _BLK, h), lambda i: (0, i, 0)),
          pl.BlockSpec((_BLK, h), lambda i: (i, 0)),
          pl.BlockSpec((_BLK, 1), lambda i: (i, 0)),
          pl.BlockSpec((1, h), lambda i: (0, 0)),
          pl.BlockSpec((1, h), lambda i: (0, 0)),
          pl.BlockSpec((1, h), lambda i: (0, 0)),
          pl.BlockSpec(W1.shape, lambda i: (0, 0)),
          pl.BlockSpec((1, h), lambda i: (0, 0)),
          pl.BlockSpec(W2.shape, lambda i: (0, 0)),
          pl.BlockSpec((1, out_dim), lambda i: (0, 0)),
      ],
      out_specs=pl.BlockSpec((1, out_dim), lambda i: (0, 0)),
      out_shape=jax.ShapeDtypeStruct((1, out_dim), jnp.float32),
      scratch_shapes=[pltpu.VMEM((1, h), jnp.float32)],
  )(s_parts, g, dis, b2, gam2, bet2, W1, b12, W2, b22)


def kernel(x, edge_index, region_ids, region_table, Wp, bp, Wl, bl,
           gamma, beta, W1, b1, W2, b2):
  n, f = x.shape
  e = edge_index.shape[1]
  ept = e // NW
  nch = ept // K
  src3 = edge_index[0].reshape(NW, nch, K)
  dst3 = edge_index[1].reshape(NW, nch, K)
  rid2 = region_ids.reshape(n, 1)

  deg_parts = _sc_deg(dst3, n)
  g, dis = _tc_embed(x, rid2, region_table, Wp, bp.reshape(1, -1),
                     Wl[0], deg_parts)
  num_layers = Wl.shape[0]
  for l in range(num_layers):
    s_parts = _sc_scatter(g, src3, dst3)
    b2_ = bl[l].reshape(1, -1)
    gam2 = gamma[l].reshape(1, -1)
    bet2 = beta[l].reshape(1, -1)
    if l < num_layers - 1:
      g = _tc_post(s_parts, g, dis, b2_, gam2, bet2, Wl[l + 1])
    else:
      out = _tc_final(s_parts, g, dis, b2_, gam2, bet2,
                      W1, b1.reshape(1, -1), W2, b2.reshape(1, -1), n)
  return out


# speedup vs baseline: 13.8277x; 13.8277x over previous
import functools
import math

import jax
import jax.numpy as jnp
from jax import lax
from jax.experimental import pallas as pl
from jax.experimental.pallas import tpu as pltpu
from jax.experimental.pallas import tpu_sc as plsc

NW = 32
NSUB = 16
K = 80


def _sc_deg(dst3, n):
  nch = dst3.shape[1]
  npad = -(-n // 128) * 128
  zeros = jnp.zeros((npad, 16), jnp.float32)
  ones = jnp.ones((K, 16), jnp.float32)
  rows_pt = npad // NSUB
  mesh = plsc.VectorSubcoreMesh(core_axis_name="c", subcore_axis_name="s")

  @functools.partial(
      pl.kernel,
      mesh=mesh,
      out_type=jax.ShapeDtypeStruct((2, npad, 16), jnp.float32),
      scratch_types=[
          pltpu.VMEM((nch, K), jnp.int32),
          pltpu.VMEM((K, 16), jnp.float32),
          pltpu.VMEM_SHARED((npad, 16), jnp.float32),
      ],
  )
  def deg_kernel(dst_hbm, zeros_hbm, ones_hbm, out_hbm, dst_v, ones_v, acc):
    c = lax.axis_index("c")
    s = lax.axis_index("s")
    wid = c * NSUB + s
    pltpu.sync_copy(dst_hbm.at[wid], dst_v)
    pltpu.sync_copy(ones_hbm, ones_v)
    base = s * rows_pt
    pltpu.sync_copy(zeros_hbm.at[pl.ds(base, rows_pt)],
                    acc.at[pl.ds(base, rows_pt)])
    plsc.subcore_barrier()

    @pl.loop(0, nch)
    def _chunk(j):
      pltpu.sync_copy(ones_v, acc.at[dst_v.at[j]], add=True)

    plsc.subcore_barrier()
    pltpu.sync_copy(acc.at[pl.ds(base, rows_pt)],
                    out_hbm.at[c, pl.ds(base, rows_pt)])

  return deg_kernel(dst3, zeros, ones)


def _sc_scatter(g, src3, dst3):
  n, h = g.shape
  nch = src3.shape[1]
  npad = -(-n // 128) * 128
  zeros = jnp.zeros((npad, h), jnp.float32)
  rows_pt = npad // NSUB
  mesh = plsc.VectorSubcoreMesh(core_axis_name="c", subcore_axis_name="s")

  @functools.partial(
      pl.kernel,
      mesh=mesh,
      out_type=jax.ShapeDtypeStruct((2, npad, h), jnp.float32),
      scratch_types=[
          pltpu.VMEM((nch, K), jnp.int32),
          pltpu.VMEM((nch, K), jnp.int32),
          pltpu.VMEM((K, h), jnp.float32),
          pltpu.VMEM_SHARED((npad, h), jnp.float32),
          pltpu.SemaphoreType.DMA,
      ],
  )
  def scat_kernel(g_hbm, src_hbm, dst_hbm, zeros_hbm, out_hbm,
                  src_v, dst_v, rows_v, acc, sem):
    c = lax.axis_index("c")
    s = lax.axis_index("s")
    wid = c * NSUB + s
    pltpu.sync_copy(src_hbm.at[wid], src_v)
    pltpu.sync_copy(dst_hbm.at[wid], dst_v)
    base = s * rows_pt
    pltpu.sync_copy(zeros_hbm.at[pl.ds(base, rows_pt)],
                    acc.at[pl.ds(base, rows_pt)])
    plsc.subcore_barrier()

    @pl.loop(0, nch)
    def _chunk(j):
      pltpu.async_copy(g_hbm.at[src_v.at[j]], rows_v, sem).wait()
      pltpu.sync_copy(rows_v, acc.at[dst_v.at[j]], add=True)

    plsc.subcore_barrier()
    pltpu.sync_copy(acc.at[pl.ds(base, rows_pt)],
                    out_hbm.at[c, pl.ds(base, rows_pt)])

  return scat_kernel(g, src3, dst3, zeros)


_BLK = 1000


def _embed_body(x_ref, rid_ref, rt_ref, wp_ref, bp_ref, w0_ref, deg_ref,
                g_ref, dis_ref):
  i = pl.program_id(0)
  f = x_ref.shape[1]
  blk = x_ref.shape[0]
  deg = deg_ref[0, :, 0:1] + deg_ref[1, :, 0:1] + 1.0
  dis = lax.rsqrt(deg)
  rtp = jnp.dot(rt_ref[...], wp_ref[f:2 * f, :],
                preferred_element_type=jnp.float32)
  rid = rid_ref[...]
  emb = jnp.zeros((blk, rtp.shape[1]), jnp.float32)
  for r in range(rt_ref.shape[0]):
    emb = emb + jnp.where(rid == r, rtp[r:r + 1, :], 0.0)
  pos = lax.broadcasted_iota(jnp.int32, (blk, f), 0).astype(jnp.float32) + (
      i * blk)
  col = lax.broadcasted_iota(jnp.int32, (blk, f), 1).astype(jnp.float32)
  half = jnp.floor(col * 0.5)
  rates = jnp.exp(half * (-2.0 / f * math.log(10000.0)))
  ang = pos * rates
  even = (half * 2.0) == col
  pe = jnp.where(even, jnp.sin(ang), jnp.cos(ang))
  h0 = (jnp.dot(x_ref[...], wp_ref[0:f, :],
                preferred_element_type=jnp.float32)
        + emb
        + jnp.dot(pe, wp_ref[2 * f:3 * f, :],
                  preferred_element_type=jnp.float32)
        + bp_ref[...])
  g_ref[...] = dis * jnp.dot(h0, w0_ref[...],
                             preferred_element_type=jnp.float32)
  dis_ref[...] = dis


def _tc_embed(x, rid2, region_table, Wp, bp2, W0, deg_parts):
  n, f = x.shape
  h = W0.shape[1]
  grid = n // _BLK
  return pl.pallas_call(
      _embed_body,
      grid=(grid,),
      in_specs=[
          pl.BlockSpec((_BLK, f), lambda i: (i, 0)),
          pl.BlockSpec((_BLK, 1), lambda i: (i, 0)),
          pl.BlockSpec(region_table.shape, lambda i: (0, 0)),
          pl.BlockSpec(Wp.shape, lambda i: (0, 0)),
          pl.BlockSpec((1, h), lambda i: (0, 0)),
          pl.BlockSpec((f, h), lambda i: (0, 0)),
          pl.BlockSpec((2, _BLK, 16), lambda i: (0, i, 0)),
      ],
      out_specs=[
          pl.BlockSpec((_BLK, h), lambda i: (i, 0)),
          pl.BlockSpec((_BLK, 1), lambda i: (i, 0)),
      ],
      out_shape=[
          jax.ShapeDtypeStruct((n, h), jnp.float32),
          jax.ShapeDtypeStruct((n, 1), jnp.float32),
      ],
  )(x, rid2, region_table, Wp, bp2, W0, deg_parts)


def _layer_h(s_ref, g_ref, dis_ref, b_ref, gam_ref, bet_ref):
  dis = dis_ref[...]
  a = dis * (s_ref[0] + s_ref[1] + g_ref[...]) + b_ref[...]
  r = jnp.maximum(a, 0.0)
  mu = jnp.mean(r, axis=-1, keepdims=True)
  d = r - mu
  var = jnp.mean(d * d, axis=-1, keepdims=True)
  return d * lax.rsqrt(var + 1e-5) * gam_ref[...] + bet_ref[...]


def _post_body(s_ref, g_ref, dis_ref, b_ref, gam_ref, bet_ref, wn_ref,
               gn_ref):
  hn = _layer_h(s_ref, g_ref, dis_ref, b_ref, gam_ref, bet_ref)
  gn_ref[...] = dis_ref[...] * jnp.dot(hn, wn_ref[...],
                                       preferred_element_type=jnp.float32)


def _tc_post(s_parts, g, dis, b2, gam2, bet2, Wn):
  n, h = g.shape
  grid = n // _BLK
  return pl.pallas_call(
      _post_body,
      grid=(grid,),
      in_specs=[
          pl.BlockSpec((2, _BLK, h), lambda i: (0, i, 0)),
          pl.BlockSpec((_BLK, h), lambda i: (i, 0)),
          pl.BlockSpec((_BLK, 1), lambda i: (i, 0)),
          pl.BlockSpec((1, h), lambda i: (0, 0)),
          pl.BlockSpec((1, h), lambda i: (0, 0)),
          pl.BlockSpec((1, h), lambda i: (0, 0)),
          pl.BlockSpec((h, h), lambda i: (0, 0)),
      ],
      out_specs=pl.BlockSpec((_BLK, h), lambda i: (i, 0)),
      out_shape=jax.ShapeDtypeStruct((n, h), jnp.float32),
  )(s_parts, g, dis, b2, gam2, bet2, Wn)


def _final_body(s_ref, g_ref, dis_ref, b_ref, gam_ref, bet_ref,
                w1_ref, b1_ref, w2_ref, b2_ref, out_ref, acc_ref, *, n):
  i = pl.program_id(0)
  hn = _layer_h(s_ref, g_ref, dis_ref, b_ref, gam_ref, bet_ref)
  part = jnp.sum(hn, axis=0, keepdims=True)

  @pl.when(i == 0)
  def _():
    acc_ref[...] = part

  @pl.when(i > 0)
  def _():
    acc_ref[...] = acc_ref[...] + part

  @pl.when(i == pl.num_programs(0) - 1)
  def _():
    pooled = acc_ref[...] * (1.0 / n)
    z = jnp.dot(pooled, w1_ref[...],
                preferred_element_type=jnp.float32) + b1_ref[...]
    hid = 0.5 * z * (1.0 + lax.erf(z * (1.0 / math.sqrt(2.0))))
    out_ref[...] = jnp.dot(hid, w2_ref[...],
                           preferred_element_type=jnp.float32) + b2_ref[...]


def _tc_final(s_parts, g, dis, b2, gam2, bet2, W1, b12, W2, b22, n):
  h = g.shape[1]
  out_dim = W2.shape[1]
  grid = n // _BLK
  return pl.pallas_call(
      functools.partial(_final_body, n=n),
      grid=(grid,),
      in_specs=[
          pl.BlockSpec((2, _BLK, h), lambda i: (0, i, 0)),
          pl.BlockSpec((_BLK, h), lambda i: (i, 0)),
          pl.BlockSpec((_BLK, 1), lambda i: (i, 0)),
          pl.BlockSpec((1, h), lambda i: (0, 0)),
          pl.BlockSpec((1, h), lambda i: (0, 0)),
          pl.BlockSpec((1, h), lambda i: (0, 0)),
          pl.BlockSpec(W1.shape, lambda i: (0, 0)),
          pl.BlockSpec((1, h), lambda i: (0, 0)),
          pl.BlockSpec(W2.shape, lambda i: (0, 0)),
          pl.BlockSpec((1, out_dim), lambda i: (0, 0)),
      ],
      out_specs=pl.BlockSpec((1, out_dim), lambda i: (0, 0)),
      out_shape=jax.ShapeDtypeStruct((1, out_dim), jnp.float32),
      scratch_shapes=[pltpu.VMEM((1, h), jnp.float32)],
  )(s_parts, g, dis, b2, gam2, bet2, W1, b12, W2, b22)


def kernel(x, edge_index, region_ids, region_table, Wp, bp, Wl, bl,
           gamma, beta, W1, b1, W2, b2):
  n, f = x.shape
  e = edge_index.shape[1]
  ept = e // NW
  nch = ept // K
  src3 = edge_index[0].reshape(NW, nch, K)
  dst3 = edge_index[1].reshape(NW, nch, K)
  rid2 = region_ids.reshape(n, 1)

  deg_parts = _sc_deg(dst3, n)
  g, dis = _tc_embed(x, rid2, region_table, Wp, bp.reshape(1, -1),
                     Wl[0], deg_parts)
  num_layers = Wl.shape[0]
  for l in range(num_layers):
    s_parts = _sc_scatter(g, src3, dst3)
    b2_ = bl[l].reshape(1, -1)
    gam2 = gamma[l].reshape(1, -1)
    bet2 = beta[l].reshape(1, -1)
    if l < num_layers - 1:
      g = _tc_post(s_parts, g, dis, b2_, gam2, bet2, Wl[l + 1])
    else:
      out = _tc_final(s_parts, g, dis, b2_, gam2, bet2,
                      W1, b1.reshape(1, -1), W2, b2.reshape(1, -1), n)
  return out
